# baseline (speedup 1.0000x reference)
import functools

import jax
import jax.numpy as jnp
from jax import lax
from jax.experimental import pallas as pl
from jax.experimental.pallas import tpu as pltpu
from jax.experimental.pallas import tpu_sc as plsc

N_NODES = 10000
N_EDGES = 320000
DIM = 128

NC = 2
NS = 16
NW = NC * NS
CH = 128
K0 = 92
K1 = 65
E0 = NS * K0 * CH
E1 = NS * K1 * CH
EPAD = E0 + E1
ZR = -(-(N_NODES + 1) // (NS * 8)) * 8
A_ROWS = ZR * NS


def _sc_segment_sum(x, src, dst, zrows):
    mesh = plsc.VectorSubcoreMesh(core_axis_name="c", subcore_axis_name="s")

    @functools.partial(
        pl.kernel,
        mesh=mesh,
        out_type=jax.ShapeDtypeStruct((NC, A_ROWS, DIM), jnp.float32),
        scratch_types=[
            pltpu.VMEM((K0, CH), jnp.int32),
            pltpu.VMEM((K0, CH), jnp.int32),
            pltpu.VMEM((CH, DIM), jnp.float32),
            pltpu.VMEM_SHARED((A_ROWS, DIM), jnp.float32),
        ],
    )
    def body(x_hbm, src_hbm, dst_hbm, z_hbm, out_hbm, src_v, dst_v, rows_v, acc):
        c = lax.axis_index("c")
        s = lax.axis_index("s")
        wid = c * NS + s

        pltpu.sync_copy(z_hbm, acc.at[pl.ds(s * ZR, ZR)])
        pltpu.sync_copy(src_hbm.at[wid], src_v)
        pltpu.sync_copy(dst_hbm.at[wid], dst_v)
        plsc.subcore_barrier()

        def step(j, _):
            pltpu.sync_copy(x_hbm.at[src_v.at[j]], rows_v)
            pltpu.sync_copy(rows_v, acc.at[dst_v.at[j]], add=True)
            return _

        kc = jnp.where(c == 0, K0, K1)
        lax.fori_loop(0, kc, step, None)
        plsc.subcore_barrier()
        pltpu.sync_copy(acc.at[pl.ds(s * ZR, ZR)],
                        out_hbm.at[c, pl.ds(s * ZR, ZR)])

    return body(x, src, dst, zrows)


def _tc_finish(x, a, w1t, w2t):
    R = 1000

    def body(x_ref, a0_ref, a1_ref, w1t_ref, w2t_ref, o_ref):
        sp = jnp.dot(x_ref[...], w1t_ref[...],
                     preferred_element_type=jnp.float32,
                     precision=lax.Precision.HIGHEST)
        np_ = jnp.dot(a0_ref[0] + a1_ref[0], w2t_ref[...],
                      preferred_element_type=jnp.float32,
                      precision=lax.Precision.HIGHEST)
        o_ref[...] = jnp.maximum(sp + np_, 0.0)

    return pl.pallas_call(
        body,
        grid=(N_NODES // R,),
        in_specs=[
            pl.BlockSpec((R, DIM), lambda i: (i, 0)),
            pl.BlockSpec((1, R, DIM), lambda i: (0, i, 0)),
            pl.BlockSpec((1, R, DIM), lambda i: (1, i, 0)),
            pl.BlockSpec((DIM, DIM), lambda i: (0, 0)),
            pl.BlockSpec((DIM, DIM), lambda i: (0, 0)),
        ],
        out_specs=pl.BlockSpec((R, DIM), lambda i: (i, 0)),
        out_shape=jax.ShapeDtypeStruct((N_NODES, DIM), jnp.float32),
    )(x, a, a, w1t, w2t)


def kernel(x, edge_index, W1, W2):
    src = edge_index[0].astype(jnp.int32)
    dst = edge_index[1].astype(jnp.int32)
    pad = EPAD - N_EDGES
    src_f = jnp.concatenate([src, jnp.zeros((pad,), jnp.int32)])
    dst_f = jnp.concatenate([dst, jnp.full((pad,), N_NODES, jnp.int32)])
    src_p = jnp.concatenate([
        src_f[:E0].reshape(NS, K0, CH),
        jnp.concatenate([src_f[E0:].reshape(NS, K1, CH),
                         jnp.zeros((NS, K0 - K1, CH), jnp.int32)], axis=1),
    ], axis=0)
    dst_p = jnp.concatenate([
        dst_f[:E0].reshape(NS, K0, CH),
        jnp.concatenate([dst_f[E0:].reshape(NS, K1, CH),
                         jnp.zeros((NS, K0 - K1, CH), jnp.int32)], axis=1),
    ], axis=0)
    zrows = jnp.zeros((ZR, DIM), jnp.float32)
    a = _sc_segment_sum(x, src_p, dst_p, zrows)
    return _tc_finish(x, a, W1.T, W2.T)


# speedup vs baseline: 1.0208x; 1.0208x over previous
import functools

import jax
import jax.numpy as jnp
from jax import lax
from jax.experimental import pallas as pl
from jax.experimental.pallas import tpu as pltpu
from jax.experimental.pallas import tpu_sc as plsc

N_NODES = 10000
N_EDGES = 320000
DIM = 128

NC = 2
NS = 16
NW = NC * NS
CH = 128
K0 = 95
K1 = 62
E0 = NS * K0 * CH
E1 = NS * K1 * CH
EPAD = E0 + E1
ZR = -(-(N_NODES + 1) // (NS * 8)) * 8
A_ROWS = ZR * NS


def _sc_segment_sum(x, src, dst, zrows):
    mesh = plsc.VectorSubcoreMesh(core_axis_name="c", subcore_axis_name="s")

    @functools.partial(
        pl.kernel,
        mesh=mesh,
        out_type=jax.ShapeDtypeStruct((NC, A_ROWS, DIM), jnp.float32),
        scratch_types=[
            pltpu.VMEM((K0, CH), jnp.int32),
            pltpu.VMEM((K0, CH), jnp.int32),
            pltpu.VMEM((CH, DIM), jnp.float32),
            pltpu.VMEM_SHARED((A_ROWS, DIM), jnp.float32),
        ],
    )
    def body(x_hbm, src_hbm, dst_hbm, z_hbm, out_hbm, src_v, dst_v, rows_v, acc):
        c = lax.axis_index("c")
        s = lax.axis_index("s")
        wid = c * NS + s

        pltpu.sync_copy(z_hbm, acc.at[pl.ds(s * ZR, ZR)])
        pltpu.sync_copy(src_hbm.at[wid], src_v)
        pltpu.sync_copy(dst_hbm.at[wid], dst_v)
        plsc.subcore_barrier()

        def step(j, _):
            pltpu.sync_copy(x_hbm.at[src_v.at[j]], rows_v)
            pltpu.sync_copy(rows_v, acc.at[dst_v.at[j]], add=True)
            return _

        kc = jnp.where(c == 0, K0, K1)
        lax.fori_loop(0, kc, step, None)
        plsc.subcore_barrier()
        pltpu.sync_copy(acc.at[pl.ds(s * ZR, ZR)],
                        out_hbm.at[c, pl.ds(s * ZR, ZR)])

    return body(x, src, dst, zrows)


def _tc_finish(x, a, w1t, w2t):
    R = 1000

    def body(x_ref, a0_ref, a1_ref, w1t_ref, w2t_ref, o_ref):
        sp = jnp.dot(x_ref[...], w1t_ref[...],
                     preferred_element_type=jnp.float32,
                     precision=lax.Precision.HIGHEST)
        np_ = jnp.dot(a0_ref[0] + a1_ref[0], w2t_ref[...],
                      preferred_element_type=jnp.float32,
                      precision=lax.Precision.HIGHEST)
        o_ref[...] = jnp.maximum(sp + np_, 0.0)

    return pl.pallas_call(
        body,
        grid=(N_NODES // R,),
        in_specs=[
            pl.BlockSpec((R, DIM), lambda i: (i, 0)),
            pl.BlockSpec((1, R, DIM), lambda i: (0, i, 0)),
            pl.BlockSpec((1, R, DIM), lambda i: (1, i, 0)),
            pl.BlockSpec((DIM, DIM), lambda i: (0, 0)),
            pl.BlockSpec((DIM, DIM), lambda i: (0, 0)),
        ],
        out_specs=pl.BlockSpec((R, DIM), lambda i: (i, 0)),
        out_shape=jax.ShapeDtypeStruct((N_NODES, DIM), jnp.float32),
    )(x, a, a, w1t, w2t)


def kernel(x, edge_index, W1, W2):
    src = edge_index[0].astype(jnp.int32)
    dst = edge_index[1].astype(jnp.int32)
    pad = EPAD - N_EDGES
    src_f = jnp.concatenate([src, jnp.zeros((pad,), jnp.int32)])
    dst_f = jnp.concatenate([dst, jnp.full((pad,), N_NODES, jnp.int32)])
    src_p = jnp.concatenate([
        src_f[:E0].reshape(NS, K0, CH),
        jnp.concatenate([src_f[E0:].reshape(NS, K1, CH),
                         jnp.zeros((NS, K0 - K1, CH), jnp.int32)], axis=1),
    ], axis=0)
    dst_p = jnp.concatenate([
        dst_f[:E0].reshape(NS, K0, CH),
        jnp.concatenate([dst_f[E0:].reshape(NS, K1, CH),
                         jnp.zeros((NS, K0 - K1, CH), jnp.int32)], axis=1),
    ], axis=0)
    zrows = jnp.zeros((ZR, DIM), jnp.float32)
    a = _sc_segment_sum(x, src_p, dst_p, zrows)
    return _tc_finish(x, a, W1.T, W2.T)


# speedup vs baseline: 1.1098x; 1.0872x over previous
import functools

import jax
import jax.numpy as jnp
from jax import lax
from jax.experimental import pallas as pl
from jax.experimental.pallas import tpu as pltpu
from jax.experimental.pallas import tpu_sc as plsc

N_NODES = 10000
N_EDGES = 320000
DIM = 128

NC = 2
NS = 16
CH = 128
NCHUNK = -(-(-(-N_EDGES // CH)) // 8) * 8
K0 = 96
C0 = NS * K0
K1B = (NCHUNK - C0) // NS // 8 * 8
K1X = (NCHUNK - C0 - NS * K1B) // 8
ZR = -(-(N_NODES + 1) // (NS * 8)) * 8
A_ROWS = ZR * NS


def _sc_segment_sum(x, src, dst, zrows):
    mesh = plsc.VectorSubcoreMesh(core_axis_name="c", subcore_axis_name="s")

    @functools.partial(
        pl.kernel,
        mesh=mesh,
        out_type=jax.ShapeDtypeStruct((NC, A_ROWS, DIM), jnp.float32),
        scratch_types=[
            pltpu.VMEM((K0, CH), jnp.int32),
            pltpu.VMEM((K0, CH), jnp.int32),
            pltpu.VMEM((CH, DIM), jnp.float32),
            pltpu.VMEM_SHARED((A_ROWS, DIM), jnp.float32),
        ],
    )
    def body(x_hbm, src_hbm, dst_hbm, z_hbm, out_hbm, src_v, dst_v, rows_v, acc):
        c = lax.axis_index("c")
        s = lax.axis_index("s")

        pltpu.sync_copy(z_hbm, acc.at[pl.ds(s * ZR, ZR)])
        @pl.when(c == 0)
        def _():
            st0 = s * K0
            pltpu.sync_copy(src_hbm.at[pl.ds(st0, K0)], src_v)
            pltpu.sync_copy(dst_hbm.at[pl.ds(st0, K0)], dst_v)

        @pl.when(c != 0)
        def _():
            st1 = C0 + s * K1B + 8 * jnp.minimum(s, K1X)
            pltpu.sync_copy(src_hbm.at[pl.ds(st1, K1B)],
                            src_v.at[pl.ds(0, K1B)])
            pltpu.sync_copy(dst_hbm.at[pl.ds(st1, K1B)],
                            dst_v.at[pl.ds(0, K1B)])

            @pl.when(s < K1X)
            def _():
                pltpu.sync_copy(src_hbm.at[pl.ds(st1 + K1B, 8)],
                                src_v.at[pl.ds(K1B, 8)])
                pltpu.sync_copy(dst_hbm.at[pl.ds(st1 + K1B, 8)],
                                dst_v.at[pl.ds(K1B, 8)])

        plsc.subcore_barrier()

        def step(j, _):
            pltpu.sync_copy(x_hbm.at[src_v.at[j]], rows_v)
            pltpu.sync_copy(rows_v, acc.at[dst_v.at[j]], add=True)
            return _

        kc = jnp.where(c == 0, K0, K1B + jnp.where(s < K1X, 8, 0))
        lax.fori_loop(0, kc, step, None)
        plsc.subcore_barrier()
        pltpu.sync_copy(acc.at[pl.ds(s * ZR, ZR)],
                        out_hbm.at[c, pl.ds(s * ZR, ZR)])

    return body(x, src, dst, zrows)


def _tc_finish(x, a, w1t, w2t):
    R = 1000

    def body(x_ref, a0_ref, a1_ref, w1t_ref, w2t_ref, o_ref):
        sp = jnp.dot(x_ref[...], w1t_ref[...],
                     preferred_element_type=jnp.float32,
                     precision=lax.Precision.HIGHEST)
        np_ = jnp.dot(a0_ref[0] + a1_ref[0], w2t_ref[...],
                      preferred_element_type=jnp.float32,
                      precision=lax.Precision.HIGHEST)
        o_ref[...] = jnp.maximum(sp + np_, 0.0)

    return pl.pallas_call(
        body,
        grid=(N_NODES // R,),
        in_specs=[
            pl.BlockSpec((R, DIM), lambda i: (i, 0)),
            pl.BlockSpec((1, R, DIM), lambda i: (0, i, 0)),
            pl.BlockSpec((1, R, DIM), lambda i: (1, i, 0)),
            pl.BlockSpec((DIM, DIM), lambda i: (0, 0)),
            pl.BlockSpec((DIM, DIM), lambda i: (0, 0)),
        ],
        out_specs=pl.BlockSpec((R, DIM), lambda i: (i, 0)),
        out_shape=jax.ShapeDtypeStruct((N_NODES, DIM), jnp.float32),
    )(x, a, a, w1t, w2t)


def kernel(x, edge_index, W1, W2):
    pad = NCHUNK * CH - N_EDGES
    src_p = jnp.concatenate(
        [edge_index[0].astype(jnp.int32), jnp.zeros((pad,), jnp.int32)]
    ).reshape(NCHUNK, CH)
    dst_p = jnp.concatenate(
        [edge_index[1].astype(jnp.int32), jnp.full((pad,), N_NODES, jnp.int32)]
    ).reshape(NCHUNK, CH)
    zrows = jnp.zeros((ZR, DIM), jnp.float32)
    a = _sc_segment_sum(x, src_p, dst_p, zrows)
    return _tc_finish(x, a, W1.T, W2.T)


# speedup vs baseline: 1.1317x; 1.0198x over previous
import functools

import jax
import jax.numpy as jnp
from jax import lax
from jax.experimental import pallas as pl
from jax.experimental.pallas import tpu as pltpu
from jax.experimental.pallas import tpu_sc as plsc

N_NODES = 10000
N_EDGES = 320000
DIM = 128

NC = 2
NS = 16
CH = 128
NCHUNK = N_EDGES // CH
K0 = 95
C0 = NS * K0
K1B = (NCHUNK - C0) // NS
K1X = NCHUNK - C0 - NS * K1B
ZR = -(-(N_NODES + 1) // (NS * 8)) * 8
A_ROWS = ZR * NS


def _sc_segment_sum(x, src, dst, zrows):
    mesh = plsc.VectorSubcoreMesh(core_axis_name="c", subcore_axis_name="s")

    @functools.partial(
        pl.kernel,
        mesh=mesh,
        out_type=jax.ShapeDtypeStruct((NC, A_ROWS, DIM), jnp.float32),
        scratch_types=[
            pltpu.VMEM((K0 * CH,), jnp.int32),
            pltpu.VMEM((K0 * CH,), jnp.int32),
            pltpu.VMEM((CH, DIM), jnp.float32),
            pltpu.VMEM_SHARED((A_ROWS, DIM), jnp.float32),
        ],
    )
    def body(x_hbm, src_hbm, dst_hbm, z_hbm, out_hbm, src_v, dst_v, rows_v, acc):
        c = lax.axis_index("c")
        s = lax.axis_index("s")

        pltpu.sync_copy(z_hbm, acc.at[pl.ds(s * ZR, ZR)])
        @pl.when(c == 0)
        def _():
            st0 = s * (K0 * CH)
            pltpu.sync_copy(src_hbm.at[pl.ds(st0, K0 * CH)], src_v)
            pltpu.sync_copy(dst_hbm.at[pl.ds(st0, K0 * CH)], dst_v)

        @pl.when(c != 0)
        def _():
            st1 = (C0 + s * K1B + jnp.minimum(s, K1X)) * CH
            pltpu.sync_copy(src_hbm.at[pl.ds(st1, K1B * CH)],
                            src_v.at[pl.ds(0, K1B * CH)])
            pltpu.sync_copy(dst_hbm.at[pl.ds(st1, K1B * CH)],
                            dst_v.at[pl.ds(0, K1B * CH)])

            @pl.when(s < K1X)
            def _():
                pltpu.sync_copy(src_hbm.at[pl.ds(st1 + K1B * CH, CH)],
                                src_v.at[pl.ds(K1B * CH, CH)])
                pltpu.sync_copy(dst_hbm.at[pl.ds(st1 + K1B * CH, CH)],
                                dst_v.at[pl.ds(K1B * CH, CH)])

        plsc.subcore_barrier()

        def step(j, _):
            pltpu.sync_copy(x_hbm.at[src_v.at[pl.ds(j * CH, CH)]], rows_v)
            pltpu.sync_copy(rows_v, acc.at[dst_v.at[pl.ds(j * CH, CH)]],
                            add=True)
            return _

        kc = jnp.where(c == 0, K0, K1B + jnp.where(s < K1X, 1, 0))
        lax.fori_loop(0, kc, step, None)
        plsc.subcore_barrier()
        pltpu.sync_copy(acc.at[pl.ds(s * ZR, ZR)],
                        out_hbm.at[c, pl.ds(s * ZR, ZR)])

    return body(x, src, dst, zrows)


def _tc_finish(x, a, w1t, w2t):
    R = 1000

    def body(x_ref, a0_ref, a1_ref, w1t_ref, w2t_ref, o_ref):
        sp = jnp.dot(x_ref[...], w1t_ref[...],
                     preferred_element_type=jnp.float32,
                     precision=lax.Precision.HIGHEST)
        np_ = jnp.dot(a0_ref[0] + a1_ref[0], w2t_ref[...],
                      preferred_element_type=jnp.float32,
                      precision=lax.Precision.HIGHEST)
        o_ref[...] = jnp.maximum(sp + np_, 0.0)

    return pl.pallas_call(
        body,
        grid=(N_NODES // R,),
        in_specs=[
            pl.BlockSpec((R, DIM), lambda i: (i, 0)),
            pl.BlockSpec((1, R, DIM), lambda i: (0, i, 0)),
            pl.BlockSpec((1, R, DIM), lambda i: (1, i, 0)),
            pl.BlockSpec((DIM, DIM), lambda i: (0, 0)),
            pl.BlockSpec((DIM, DIM), lambda i: (0, 0)),
        ],
        out_specs=pl.BlockSpec((R, DIM), lambda i: (i, 0)),
        out_shape=jax.ShapeDtypeStruct((N_NODES, DIM), jnp.float32),
    )(x, a, a, w1t, w2t)


def kernel(x, edge_index, W1, W2):
    src_p = edge_index[0].astype(jnp.int32)
    dst_p = edge_index[1].astype(jnp.int32)
    zrows = jnp.zeros((ZR, DIM), jnp.float32)
    a = _sc_segment_sum(x, src_p, dst_p, zrows)
    return _tc_finish(x, a, W1.T, W2.T)
